# Initial kernel scaffold; baseline (speedup 1.0000x reference)
import jax
import jax.numpy as jnp
from jax.experimental import pallas as pl


def kernel(word_seq_, char_seq_, pos_seq_, sample_ix, word_table, char_table, pos_table, W1, b1, W2, b2, Wt, bt):
    raise NotImplementedError("write your pallas kernel here")


# speedup vs baseline: 14.9932x; 14.9932x over previous
import jax
import jax.numpy as jnp
import numpy as np
from jax.experimental import pallas as pl
from jax.experimental.pallas import tpu as pltpu

S = 64
D = 704
H = 1024
L = 256
NEG = -1e30


def _body(h_ref, w1_ref, b1_ref, w2_ref, b2_ref, wt0_ref, bt0_ref, out_ref):
    h = h_ref[:]
    A = jnp.dot(h, w1_ref[0:D, :], preferred_element_type=jnp.float32)
    Bm = jnp.dot(h, w1_ref[D:2 * D, :], preferred_element_type=jnp.float32)
    Hc = jnp.dot(h, w1_ref[2 * D:3 * D, :], preferred_element_type=jnp.float32)
    row2 = jax.lax.broadcasted_iota(jnp.int32, (2 * S, S), 0)
    col2 = jax.lax.broadcasted_iota(jnp.int32, (2 * S, S), 1)
    ltri = (col2 < row2).astype(jnp.float32)
    Cc = jnp.dot(ltri, Hc, preferred_element_type=jnp.float32)
    Ci = Cc[0:S, :]

    b1v = b1_ref[:]
    b2v = b2_ref[:]
    wt0 = wt0_ref[:]
    w2 = w2_ref[:]
    rows64 = jax.lax.broadcasted_iota(jnp.int32, (S, 1), 0)
    col0 = (jax.lax.broadcasted_iota(jnp.int32, (1, L), 1) == 0)
    lane64 = jax.lax.broadcasted_iota(jnp.int32, (S, 2 * S), 1)
    lane128 = jax.lax.broadcasted_iota(jnp.int32, (2 * S, 2 * S), 1)

    def span_step(ln, carry):
        gold_acc, tag_acc, SC = carry
        inv = 1.0 / ln.astype(jnp.float32)
        Bsh = pltpu.roll(Bm, S + 1 - ln, axis=0)
        Cj = pltpu.roll(Cc, 2 * S - ln, axis=0)[0:S, :]
        hid = jnp.maximum(A + Bsh + (Cj - Ci) * inv + b1v, 0.0)
        feats = jnp.dot(hid, w2, preferred_element_type=jnp.float32) + b2v
        gmask = jnp.logical_and(rows64 == (S - ln), col0)
        feats = feats - gmask.astype(jnp.float32)
        gold_acc = gold_acc + jnp.sum(jnp.where(gmask, feats, 0.0))
        scores = jnp.max(feats, axis=1, keepdims=True)
        SC = jnp.where(lane64 == ln, scores, SC)
        tagv = jnp.dot(hid, wt0, preferred_element_type=jnp.float32)
        tag_acc = tag_acc + jnp.sum(jnp.where(rows64 == (S - ln), tagv, 0.0))
        return gold_acc, tag_acc, SC

    gold_acc, tag_acc, SC = jax.lax.fori_loop(
        2, S + 1, span_step,
        (jnp.float32(0.0), jnp.float32(0.0), jnp.zeros((S, 2 * S), jnp.float32)))

    Linit = jnp.zeros((S, 2 * S), jnp.float32)
    Rinit = jnp.where(lane128 == S - 1, 0.0,
                      jnp.full((2 * S, 2 * S), NEG, jnp.float32))

    def cyk_step(ln, carry):
        Lc, Rc = carry
        Rr = pltpu.roll(Rc, 2 * S - ln, axis=0)
        Rrr = pltpu.roll(Rr, S + ln, axis=1)
        win = Lc[:, 0:S] + Rrr[0:S, 0:S]
        split = jnp.max(win, axis=1, keepdims=True)
        valsc = jnp.sum(jnp.where(lane64 == ln, SC, 0.0), axis=1, keepdims=True)
        val = valsc + split
        Lc = jnp.where(lane64 == ln, val, Lc)
        valp = jnp.concatenate([val, jnp.zeros((S, 1), jnp.float32)], axis=0)
        valr = pltpu.roll(valp, ln, axis=0)
        Rc = jnp.where(lane128 == S - ln, valr, Rc)
        return Lc, Rc

    Lfin, _ = jax.lax.fori_loop(2, S + 1, cyk_step, (Linit, Rinit))

    rows64b = jax.lax.broadcasted_iota(jnp.int32, (S, 2 * S), 0)
    pred = jnp.sum(jnp.where(jnp.logical_and(rows64b == 0, lane64 == S),
                             Lfin, 0.0))
    loss_global = jnp.maximum(pred - gold_acc, 0.0) / (S - 1.0)
    nll_tag = -(tag_acc / (S - 1.0) + bt0_ref[0, 0])
    out_ref[:] = jnp.full((1, 1), nll_tag + loss_global, jnp.float32)


def kernel(word_seq_, char_seq_, pos_seq_, sample_ix, word_table, char_table,
           pos_table, W1, b1, W2, b2, Wt, bt):
    w = word_table[word_seq_]
    c = jnp.mean(char_table[char_seq_], axis=1)
    p = pos_table[pos_seq_]
    h = jnp.concatenate([w, c, p], axis=-1)

    out = pl.pallas_call(
        _body,
        out_shape=jax.ShapeDtypeStruct((1, 1), jnp.float32),
    )(h, W1, b1.reshape(1, H), W2, b2.reshape(1, L), Wt[:, 0:1],
      bt[0].reshape(1, 1))
    return out[0, 0]


# speedup vs baseline: 22.2321x; 1.4828x over previous
import jax
import jax.numpy as jnp
import numpy as np
from jax.experimental import pallas as pl
from jax.experimental.pallas import tpu as pltpu

S = 64
D = 704
H = 1024
L = 256
NEG = -1e30


def _body(h_ref, w1_ref, b1_ref, w2_ref, b2_ref, wt0_ref, bt0_ref, out_ref):
    h = h_ref[:]
    A = jnp.dot(h, w1_ref[0:D, :], preferred_element_type=jnp.float32)
    Bm = jnp.dot(h, w1_ref[D:2 * D, :], preferred_element_type=jnp.float32)
    Hc = jnp.dot(h, w1_ref[2 * D:3 * D, :], preferred_element_type=jnp.float32)
    rowB = jax.lax.broadcasted_iota(jnp.int32, (2 * S, S), 0)
    colB = jax.lax.broadcasted_iota(jnp.int32, (2 * S, S), 1)
    ltri = (colB < rowB).astype(jnp.float32)
    Cc = jnp.dot(ltri, Hc, preferred_element_type=jnp.float32)
    Ci = Cc[0:S, :]
    A2 = jnp.concatenate([A, A], axis=0)
    Ci2 = jnp.concatenate([Ci, Ci], axis=0)

    b1v = b1_ref[:]
    b2v = b2_ref[:]
    wt0 = wt0_ref[:]
    w2 = w2_ref[:]
    rows64 = jax.lax.broadcasted_iota(jnp.int32, (S, 1), 0)
    rows128 = jax.lax.broadcasted_iota(jnp.int32, (2 * S, 1), 0)
    col0 = (jax.lax.broadcasted_iota(jnp.int32, (1, L), 1) == 0)
    lane64 = jax.lax.broadcasted_iota(jnp.int32, (S, 2 * S), 1)
    lane128 = jax.lax.broadcasted_iota(jnp.int32, (2 * S, 2 * S), 1)
    rowC = jax.lax.broadcasted_iota(jnp.int32, (2 * S, 2 * S), 0)
    colC = lane128
    rmodB = jnp.bitwise_and(rowB, S - 1)
    rmodC = jnp.bitwise_and(rowC, S - 1)
    halfB = (rowB >= S).astype(jnp.int32)
    halfC = (rowC >= S).astype(jnp.int32)

    def pair_step(p, carry):
        gold_acc, tag_acc, SC = carry
        ln1 = p + 2
        lnB = ln1 + 31 * halfB
        PB = (colB == rmodB + lnB - 1).astype(jnp.float32)
        lnC = ln1 + 31 * halfC
        PC = (colC == rmodC + lnC).astype(jnp.float32)
        Bsh = jnp.dot(PB, Bm, preferred_element_type=jnp.float32)
        Cj = jnp.dot(PC, Cc, preferred_element_type=jnp.float32)
        ln1f = ln1.astype(jnp.float32)
        inv2 = jnp.where(rows128 < S, 1.0 / ln1f, 1.0 / (ln1f + 31.0))
        hid = jnp.maximum(A2 + Bsh + (Cj - Ci2) * inv2 + b1v, 0.0)
        feats = jnp.dot(hid, w2, preferred_element_type=jnp.float32) + b2v
        rowm = jnp.logical_or(rows128 == S - ln1, rows128 == 97 - ln1)
        gmask = jnp.logical_and(rowm, col0)
        feats = feats - gmask.astype(jnp.float32)
        gold_acc = gold_acc + jnp.sum(jnp.where(gmask, feats, 0.0))
        scores = jnp.max(feats, axis=1, keepdims=True)
        SC = jnp.where(lane64 == ln1, scores[0:S], SC)
        SC = jnp.where(lane64 == ln1 + 31, scores[S:2 * S], SC)
        tagv = jnp.dot(hid, wt0, preferred_element_type=jnp.float32)
        tag_acc = tag_acc + jnp.sum(jnp.where(rowm, tagv, 0.0))
        return gold_acc, tag_acc, SC

    gold_acc, tag_acc, SC = jax.lax.fori_loop(
        0, 31, pair_step,
        (jnp.float32(0.0), jnp.float32(0.0), jnp.zeros((S, 2 * S), jnp.float32)))

    Bsh64 = pltpu.roll(Bm, 1, axis=0)
    hid64 = jnp.maximum(A + Bsh64 + (Cc[S:2 * S] - Ci) * (1.0 / S) + b1v, 0.0)
    feats64 = jnp.dot(hid64, w2, preferred_element_type=jnp.float32) + b2v
    gmask64 = jnp.logical_and(rows64 == 0, col0)
    feats64 = feats64 - gmask64.astype(jnp.float32)
    gold_acc = gold_acc + jnp.sum(jnp.where(gmask64, feats64, 0.0))
    scores64 = jnp.max(feats64, axis=1, keepdims=True)
    SC = jnp.where(lane64 == S, scores64, SC)
    tagv64 = jnp.dot(hid64, wt0, preferred_element_type=jnp.float32)
    tag_acc = tag_acc + jnp.sum(jnp.where(rows64 == 0, tagv64, 0.0))

    Linit = jnp.zeros((S, 2 * S), jnp.float32)
    Rinit = jnp.where(lane128 == S - 1, 0.0,
                      jnp.full((2 * S, 2 * S), NEG, jnp.float32))

    def cyk_step(ln, carry):
        Lc, Rc = carry
        Rr = pltpu.roll(Rc, 2 * S - ln, axis=0)
        Rrr = pltpu.roll(Rr, S + ln, axis=1)
        win = Lc[:, 0:S] + Rrr[0:S, 0:S]
        split = jnp.max(win, axis=1, keepdims=True)
        valsc = jnp.sum(jnp.where(lane64 == ln, SC, 0.0), axis=1, keepdims=True)
        val = valsc + split
        Lc = jnp.where(lane64 == ln, val, Lc)
        valp = jnp.concatenate([val, jnp.zeros((S, 1), jnp.float32)], axis=0)
        valr = pltpu.roll(valp, ln, axis=0)
        Rc = jnp.where(lane128 == S - ln, valr, Rc)
        return Lc, Rc

    Lfin, _ = jax.lax.fori_loop(2, S + 1, cyk_step, (Linit, Rinit))

    rows64b = jax.lax.broadcasted_iota(jnp.int32, (S, 2 * S), 0)
    pred = jnp.sum(jnp.where(jnp.logical_and(rows64b == 0, lane64 == S),
                             Lfin, 0.0))
    loss_global = jnp.maximum(pred - gold_acc, 0.0) / (S - 1.0)
    nll_tag = -(tag_acc / (S - 1.0) + bt0_ref[0, 0])
    out_ref[:] = jnp.full((1, 1), nll_tag + loss_global, jnp.float32)


def kernel(word_seq_, char_seq_, pos_seq_, sample_ix, word_table, char_table,
           pos_table, W1, b1, W2, b2, Wt, bt):
    w = word_table[word_seq_]
    c = jnp.mean(char_table[char_seq_], axis=1)
    p = pos_table[pos_seq_]
    h = jnp.concatenate([w, c, p], axis=-1)

    out = pl.pallas_call(
        _body,
        out_shape=jax.ShapeDtypeStruct((1, 1), jnp.float32),
    )(h, W1, b1.reshape(1, H), W2, b2.reshape(1, L), Wt[:, 0:1],
      bt[0].reshape(1, 1))
    return out[0, 0]
